# baseline (speedup 1.0000x reference)
import functools

import jax
import jax.numpy as jnp
from jax import lax
from jax.experimental import pallas as pl
from jax.experimental.pallas import tpu as pltpu


def _fused_kernel(x_ref, wqkv_ref, wo_ref, o_ref, wqkv_bf, wo_bf, vext, e_ref,
                  *, num_heads, dk, dv, L, eps, scale):
    nq = num_heads * dk
    blk = dv + 128

    @pl.when(pl.program_id(0) == 0)
    def _init_constants():
        w = wqkv_ref[...]
        sc = jnp.where(
            lax.broadcasted_iota(jnp.int32, w.shape, 1) < nq, scale, 1.0)
        wqkv_bf[...] = (w * sc).astype(jnp.bfloat16)
        wo_bf[...] = wo_ref[...].astype(jnp.bfloat16)
        vcol = lax.broadcasted_iota(jnp.int32, (L, num_heads * blk), 1) % blk
        vext[...] = (vcol == dv).astype(jnp.bfloat16)
        erow = lax.broadcasted_iota(jnp.int32, (num_heads, num_heads * dv), 0)
        ecol = lax.broadcasted_iota(jnp.int32, (num_heads, num_heads * dv), 1)
        e_ref[...] = (ecol // dv == erow).astype(jnp.bfloat16)

    x = x_ref[0].astype(jnp.bfloat16)
    qkv = jnp.dot(x, wqkv_bf[...], preferred_element_type=jnp.float32)

    q = qkv[:, :nq].astype(jnp.bfloat16)
    k = qkv[:, nq:2 * nq].astype(jnp.bfloat16)
    v = qkv[:, 2 * nq:].astype(jnp.bfloat16)
    for h in range(num_heads):
        vext[:, h * blk:h * blk + dv] = v[:, h * dv:(h + 1) * dv]

    row = lax.broadcasted_iota(jnp.int32, (L, L), 0)
    col = lax.broadcasted_iota(jnp.int32, (L, L), 1)
    causal = col <= row

    o_parts, z_parts = [], []
    for h in range(num_heads):
        qh = q[:, h * dk:(h + 1) * dk]
        kh = k[:, h * dk:(h + 1) * dk]
        s = lax.dot_general(qh, kh, (((1,), (1,)), ((), ())),
                            preferred_element_type=jnp.float32)
        sb = s.astype(jnp.bfloat16)
        attn = 1.0 + sb + 0.5 * (sb * sb)
        attn = jnp.where(causal, attn, 0.0)
        oz = jnp.dot(attn, vext[:, h * blk:(h + 1) * blk],
                     preferred_element_type=jnp.float32)
        o_parts.append(oz[:, :dv])
        z_parts.append(oz[:, dv:dv + 1])
    z8 = jnp.concatenate(z_parts, axis=-1)
    inv8 = (1.0 / (z8 + eps)).astype(jnp.bfloat16)
    invbc = jnp.dot(inv8, e_ref[...],
                    preferred_element_type=jnp.float32)
    o_cat = jnp.concatenate(o_parts, axis=-1)
    o_norm = (o_cat * invbc).astype(jnp.bfloat16)

    o_ref[0] = jnp.dot(o_norm, wo_bf[...],
                       preferred_element_type=jnp.float32).astype(o_ref.dtype)


def kernel(Wqkv, Wo, x):
    B, L, D = x.shape
    num_heads = 8
    dk = 16
    nq = num_heads * dk
    nv = Wo.shape[0]
    dv = nv // num_heads
    eps = 1e-6
    scale = float(dk) ** -0.5

    body = functools.partial(_fused_kernel, num_heads=num_heads, dk=dk, dv=dv,
                             L=L, eps=eps, scale=scale)
    return pl.pallas_call(
        body,
        out_shape=jax.ShapeDtypeStruct((B, L, D), x.dtype),
        grid_spec=pltpu.PrefetchScalarGridSpec(
            num_scalar_prefetch=0,
            grid=(B,),
            in_specs=[
                pl.BlockSpec((1, L, D), lambda b: (b, 0, 0)),
                pl.BlockSpec((D, 2 * nq + nv), lambda b: (0, 0)),
                pl.BlockSpec((nv, D), lambda b: (0, 0)),
            ],
            out_specs=pl.BlockSpec((1, L, D), lambda b: (b, 0, 0)),
            scratch_shapes=[
                pltpu.VMEM((D, 2 * nq + nv), jnp.bfloat16),
                pltpu.VMEM((nv, D), jnp.bfloat16),
                pltpu.VMEM((L, num_heads * (dv + 128)), jnp.bfloat16),
                pltpu.VMEM((num_heads, nv), jnp.bfloat16),
            ],
        ),
        compiler_params=pltpu.CompilerParams(
            dimension_semantics=("arbitrary",)),
    )(x, Wqkv, Wo)


# speedup vs baseline: 1.0005x; 1.0005x over previous
import functools

import jax
import jax.numpy as jnp
from jax import lax
from jax.experimental import pallas as pl
from jax.experimental.pallas import tpu as pltpu


def _fused_kernel(x_ref, wqkv_ref, wo_ref, o_ref, wqkv_bf, wo_bf, e_ref,
                  *, num_heads, dk, dv, L, eps, scale):
    nq = num_heads * dk

    @pl.when(pl.program_id(0) == 0)
    def _init_constants():
        w = wqkv_ref[...]
        sc = jnp.where(
            lax.broadcasted_iota(jnp.int32, w.shape, 1) < nq, scale, 1.0)
        wqkv_bf[...] = (w * sc).astype(jnp.bfloat16)
        wo_bf[...] = wo_ref[...].astype(jnp.bfloat16)
        erow = lax.broadcasted_iota(jnp.int32, (num_heads, num_heads * dv), 0)
        ecol = lax.broadcasted_iota(jnp.int32, (num_heads, num_heads * dv), 1)
        e_ref[...] = (ecol // dv == erow).astype(jnp.bfloat16)

    x = x_ref[0].astype(jnp.bfloat16)
    qkv = jnp.dot(x, wqkv_bf[...], preferred_element_type=jnp.float32)

    q = qkv[:, :nq].astype(jnp.bfloat16)
    k = qkv[:, nq:2 * nq].astype(jnp.bfloat16)
    v = qkv[:, 2 * nq:].astype(jnp.bfloat16)

    row = lax.broadcasted_iota(jnp.int32, (L, L), 0)
    col = lax.broadcasted_iota(jnp.int32, (L, L), 1)
    causal = col <= row
    onescol = (lax.broadcasted_iota(jnp.int32, (L, 128), 1) == 0
               ).astype(jnp.bfloat16)

    o_parts, z_parts = [], []
    for h in range(num_heads):
        qh = q[:, h * dk:(h + 1) * dk]
        kh = k[:, h * dk:(h + 1) * dk]
        vh = v[:, h * dv:(h + 1) * dv]
        s = lax.dot_general(qh, kh, (((1,), (1,)), ((), ())),
                            preferred_element_type=jnp.float32)
        sb = s.astype(jnp.bfloat16)
        attn = 1.0 + sb + 0.5 * (sb * sb)
        attn = jnp.where(causal, attn, 0.0)
        vhx = jnp.concatenate([vh, onescol], axis=-1)
        oz = jnp.dot(attn, vhx,
                     preferred_element_type=jnp.float32)
        o_parts.append(oz[:, :dv])
        z_parts.append(oz[:, dv:dv + 1])
    z8 = jnp.concatenate(z_parts, axis=-1)
    inv8 = (1.0 / (z8 + eps)).astype(jnp.bfloat16)
    invbc = jnp.dot(inv8, e_ref[...],
                    preferred_element_type=jnp.float32)
    o_cat = jnp.concatenate(o_parts, axis=-1)
    o_norm = (o_cat * invbc).astype(jnp.bfloat16)

    o_ref[0] = jnp.dot(o_norm, wo_bf[...],
                       preferred_element_type=jnp.float32).astype(o_ref.dtype)


def kernel(Wqkv, Wo, x):
    B, L, D = x.shape
    num_heads = 8
    dk = 16
    nq = num_heads * dk
    nv = Wo.shape[0]
    dv = nv // num_heads
    eps = 1e-6
    scale = float(dk) ** -0.5

    body = functools.partial(_fused_kernel, num_heads=num_heads, dk=dk, dv=dv,
                             L=L, eps=eps, scale=scale)
    return pl.pallas_call(
        body,
        out_shape=jax.ShapeDtypeStruct((B, L, D), x.dtype),
        grid_spec=pltpu.PrefetchScalarGridSpec(
            num_scalar_prefetch=0,
            grid=(B,),
            in_specs=[
                pl.BlockSpec((1, L, D), lambda b: (b, 0, 0)),
                pl.BlockSpec((D, 2 * nq + nv), lambda b: (0, 0)),
                pl.BlockSpec((nv, D), lambda b: (0, 0)),
            ],
            out_specs=pl.BlockSpec((1, L, D), lambda b: (b, 0, 0)),
            scratch_shapes=[
                pltpu.VMEM((D, 2 * nq + nv), jnp.bfloat16),
                pltpu.VMEM((nv, D), jnp.bfloat16),
                pltpu.VMEM((num_heads, nv), jnp.bfloat16),
            ],
        ),
        compiler_params=pltpu.CompilerParams(
            dimension_semantics=("arbitrary",)),
    )(x, Wqkv, Wo)


# speedup vs baseline: 1.0162x; 1.0157x over previous
import functools

import jax
import jax.numpy as jnp
from jax import lax
from jax.experimental import pallas as pl
from jax.experimental.pallas import tpu as pltpu


def _fused_kernel(x_ref, wqkv_ref, wo_ref, o_ref, wqkv_bf, wo_bf, e_ref,
                  *, num_heads, dk, dv, L, eps, scale):
    nq = num_heads * dk

    @pl.when(pl.program_id(0) == 0)
    def _init_constants():
        w = wqkv_ref[...]
        sc = jnp.where(
            lax.broadcasted_iota(jnp.int32, w.shape, 1) < nq, scale, 1.0)
        wqkv_bf[...] = (w * sc).astype(jnp.bfloat16)
        wo_bf[...] = wo_ref[...].astype(jnp.bfloat16)
        erow = lax.broadcasted_iota(jnp.int32, (num_heads, num_heads * dv), 0)
        ecol = lax.broadcasted_iota(jnp.int32, (num_heads, num_heads * dv), 1)
        e_ref[...] = (ecol // dv == erow).astype(jnp.bfloat16)

    row = lax.broadcasted_iota(jnp.int32, (L, L), 0)
    col = lax.broadcasted_iota(jnp.int32, (L, L), 1)
    causal = col <= row
    onescol = (lax.broadcasted_iota(jnp.int32, (L, 128), 1) == 0
               ).astype(jnp.bfloat16)

    nb = x_ref.shape[0]
    for i in range(nb):
        x = x_ref[i].astype(jnp.bfloat16)
        qkv = jnp.dot(x, wqkv_bf[...], preferred_element_type=jnp.float32)

        q = qkv[:, :nq].astype(jnp.bfloat16)
        k = qkv[:, nq:2 * nq].astype(jnp.bfloat16)
        v = qkv[:, 2 * nq:].astype(jnp.bfloat16)

        o_parts, z_parts = [], []
        for h in range(num_heads):
            qh = q[:, h * dk:(h + 1) * dk]
            kh = k[:, h * dk:(h + 1) * dk]
            vh = v[:, h * dv:(h + 1) * dv]
            s = lax.dot_general(qh, kh, (((1,), (1,)), ((), ())),
                                preferred_element_type=jnp.float32)
            sb = s.astype(jnp.bfloat16)
            attn = 1.0 + sb + 0.5 * (sb * sb)
            attn = jnp.where(causal, attn, 0.0)
            vhx = jnp.concatenate([vh, onescol], axis=-1)
            oz = jnp.dot(attn, vhx,
                         preferred_element_type=jnp.float32)
            o_parts.append(oz[:, :dv])
            z_parts.append(oz[:, dv:dv + 1])
        z8 = jnp.concatenate(z_parts, axis=-1)
        inv8 = (1.0 / (z8 + eps)).astype(jnp.bfloat16)
        invbc = jnp.dot(inv8, e_ref[...],
                        preferred_element_type=jnp.float32)
        o_cat = jnp.concatenate(o_parts, axis=-1)
        o_norm = (o_cat * invbc).astype(jnp.bfloat16)

        o_ref[i] = jnp.dot(o_norm, wo_bf[...],
                           preferred_element_type=jnp.float32).astype(o_ref.dtype)


def kernel(Wqkv, Wo, x):
    B, L, D = x.shape
    num_heads = 8
    dk = 16
    nq = num_heads * dk
    nv = Wo.shape[0]
    dv = nv // num_heads
    eps = 1e-6
    scale = float(dk) ** -0.5

    body = functools.partial(_fused_kernel, num_heads=num_heads, dk=dk, dv=dv,
                             L=L, eps=eps, scale=scale)
    return pl.pallas_call(
        body,
        out_shape=jax.ShapeDtypeStruct((B, L, D), x.dtype),
        grid_spec=pltpu.PrefetchScalarGridSpec(
            num_scalar_prefetch=0,
            grid=(B // 2,),
            in_specs=[
                pl.BlockSpec((2, L, D), lambda b: (b, 0, 0)),
                pl.BlockSpec((D, 2 * nq + nv), lambda b: (0, 0)),
                pl.BlockSpec((nv, D), lambda b: (0, 0)),
            ],
            out_specs=pl.BlockSpec((2, L, D), lambda b: (b, 0, 0)),
            scratch_shapes=[
                pltpu.VMEM((D, 2 * nq + nv), jnp.bfloat16),
                pltpu.VMEM((nv, D), jnp.bfloat16),
                pltpu.VMEM((num_heads, nv), jnp.bfloat16),
            ],
        ),
        compiler_params=pltpu.CompilerParams(
            dimension_semantics=("arbitrary",)),
    )(x, Wqkv, Wo)


# speedup vs baseline: 1.0750x; 1.0579x over previous
import functools

import jax
import jax.numpy as jnp
from jax import lax
from jax.experimental import pallas as pl
from jax.experimental.pallas import tpu as pltpu


def _fused_kernel(x_ref, wqkv_ref, wo_ref, o_ref, wqkv_bf, wo_bf, *,
                  num_heads, dk, dv, L, eps, scale):
    nq = num_heads * dk

    @pl.when(pl.program_id(0) == 0)
    def _cast_weights():
        w = wqkv_ref[...]
        sc = jnp.where(
            lax.broadcasted_iota(jnp.int32, w.shape, 1) < nq, scale, 1.0)
        wqkv_bf[...] = (w * sc).astype(jnp.bfloat16)
        wo_bf[...] = wo_ref[...].astype(jnp.bfloat16)

    x = x_ref[0].astype(jnp.bfloat16)
    qkv = jnp.dot(x, wqkv_bf[...], preferred_element_type=jnp.float32)

    q = qkv[:, :nq].astype(jnp.bfloat16)
    k = qkv[:, nq:2 * nq].astype(jnp.bfloat16)
    v = qkv[:, 2 * nq:].astype(jnp.bfloat16)

    row = lax.broadcasted_iota(jnp.int32, (L, L), 0)
    col = lax.broadcasted_iota(jnp.int32, (L, L), 1)
    causal = col <= row

    o_parts = []
    for h in range(num_heads):
        qh = q[:, h * dk:(h + 1) * dk]
        kh = k[:, h * dk:(h + 1) * dk]
        vh = v[:, h * dv:(h + 1) * dv]
        s = lax.dot_general(qh, kh, (((1,), (1,)), ((), ())),
                            preferred_element_type=jnp.float32)
        attn = 1.0 + s + 0.5 * (s * s)
        attn = jnp.where(causal, attn, 0.0)
        z = jnp.sum(attn, axis=-1, keepdims=True)
        oh = jnp.dot(attn.astype(jnp.bfloat16), vh,
                     preferred_element_type=jnp.float32)
        o_parts.append(oh * (1.0 / (z + eps)))
    o_norm = jnp.concatenate(o_parts, axis=-1).astype(jnp.bfloat16)

    o_ref[0] = jnp.dot(o_norm, wo_bf[...],
                       preferred_element_type=jnp.float32).astype(o_ref.dtype)


def kernel(Wqkv, Wo, x):
    B, L, D = x.shape
    num_heads = 8
    dk = 16
    nq = num_heads * dk
    nv = Wo.shape[0]
    dv = nv // num_heads
    eps = 1e-6
    scale = float(dk) ** -0.5

    body = functools.partial(_fused_kernel, num_heads=num_heads, dk=dk, dv=dv,
                             L=L, eps=eps, scale=scale)
    return pl.pallas_call(
        body,
        out_shape=jax.ShapeDtypeStruct((B, L, D), x.dtype),
        grid_spec=pltpu.PrefetchScalarGridSpec(
            num_scalar_prefetch=0,
            grid=(B,),
            in_specs=[
                pl.BlockSpec((1, L, D), lambda b: (b, 0, 0)),
                pl.BlockSpec((D, 2 * nq + nv), lambda b: (0, 0)),
                pl.BlockSpec((nv, D), lambda b: (0, 0)),
            ],
            out_specs=pl.BlockSpec((1, L, D), lambda b: (b, 0, 0)),
            scratch_shapes=[
                pltpu.VMEM((D, 2 * nq + nv), jnp.bfloat16),
                pltpu.VMEM((nv, D), jnp.bfloat16),
            ],
        ),
        compiler_params=pltpu.CompilerParams(
            dimension_semantics=("arbitrary",)),
    )(x, Wqkv, Wo)


# speedup vs baseline: 1.0956x; 1.0192x over previous
import functools

import jax
import jax.numpy as jnp
from jax import lax
from jax.experimental import pallas as pl
from jax.experimental.pallas import tpu as pltpu


def _fused_kernel(x_ref, wqkv_ref, wo_ref, o_ref, wqkv_bf, wo_bf, *,
                  num_heads, dk, dv, L, eps, scale):
    nq = num_heads * dk

    @pl.when(pl.program_id(0) == 0)
    def _cast_weights():
        w = wqkv_ref[...]
        sc = jnp.where(
            lax.broadcasted_iota(jnp.int32, w.shape, 1) < nq, scale, 1.0)
        wqkv_bf[...] = (w * sc).astype(jnp.bfloat16)
        wo_bf[...] = wo_ref[...].astype(jnp.bfloat16)

    x = x_ref[0].astype(jnp.bfloat16)
    qkv = jnp.dot(x, wqkv_bf[...], preferred_element_type=jnp.float32)

    q = qkv[:, :nq].astype(jnp.bfloat16)
    k = qkv[:, nq:2 * nq].astype(jnp.bfloat16)
    v = qkv[:, 2 * nq:].astype(jnp.bfloat16)

    H2 = L // 2
    rowm = lax.broadcasted_iota(jnp.int32, (H2, H2), 0)
    colm = lax.broadcasted_iota(jnp.int32, (H2, H2), 1)
    tri = colm <= rowm
    causal_b = jnp.concatenate(
        [jnp.ones((H2, H2), jnp.bool_), tri], axis=-1)

    o_parts = []
    for h in range(num_heads):
        qh = q[:, h * dk:(h + 1) * dk]
        kh = k[:, h * dk:(h + 1) * dk]
        vh = v[:, h * dv:(h + 1) * dv]
        qa, qb = qh[:H2], qh[H2:]
        ka = kh[:H2]
        sa = lax.dot_general(qa, ka, (((1,), (1,)), ((), ())),
                             preferred_element_type=jnp.float32)
        attna = 1.0 + sa + 0.5 * (sa * sa)
        attna = jnp.where(tri, attna, 0.0)
        za = jnp.sum(attna, axis=-1, keepdims=True)
        oa = jnp.dot(attna.astype(jnp.bfloat16), vh[:H2],
                     preferred_element_type=jnp.float32)
        sb = lax.dot_general(qb, kh, (((1,), (1,)), ((), ())),
                             preferred_element_type=jnp.float32)
        attnb = 1.0 + sb + 0.5 * (sb * sb)
        attnb = jnp.where(causal_b, attnb, 0.0)
        zb = jnp.sum(attnb, axis=-1, keepdims=True)
        ob = jnp.dot(attnb.astype(jnp.bfloat16), vh,
                     preferred_element_type=jnp.float32)
        oh = jnp.concatenate(
            [oa * (1.0 / (za + eps)), ob * (1.0 / (zb + eps))], axis=0)
        o_parts.append(oh)
    o_norm = jnp.concatenate(o_parts, axis=-1).astype(jnp.bfloat16)

    o_ref[0] = jnp.dot(o_norm, wo_bf[...],
                       preferred_element_type=jnp.float32).astype(o_ref.dtype)


def kernel(Wqkv, Wo, x):
    B, L, D = x.shape
    num_heads = 8
    dk = 16
    nq = num_heads * dk
    nv = Wo.shape[0]
    dv = nv // num_heads
    eps = 1e-6
    scale = float(dk) ** -0.5

    body = functools.partial(_fused_kernel, num_heads=num_heads, dk=dk, dv=dv,
                             L=L, eps=eps, scale=scale)
    return pl.pallas_call(
        body,
        out_shape=jax.ShapeDtypeStruct((B, L, D), x.dtype),
        grid_spec=pltpu.PrefetchScalarGridSpec(
            num_scalar_prefetch=0,
            grid=(B,),
            in_specs=[
                pl.BlockSpec((1, L, D), lambda b: (b, 0, 0)),
                pl.BlockSpec((D, 2 * nq + nv), lambda b: (0, 0)),
                pl.BlockSpec((nv, D), lambda b: (0, 0)),
            ],
            out_specs=pl.BlockSpec((1, L, D), lambda b: (b, 0, 0)),
            scratch_shapes=[
                pltpu.VMEM((D, 2 * nq + nv), jnp.bfloat16),
                pltpu.VMEM((nv, D), jnp.bfloat16),
            ],
        ),
        compiler_params=pltpu.CompilerParams(
            dimension_semantics=("arbitrary",)),
    )(x, Wqkv, Wo)


# speedup vs baseline: 1.0982x; 1.0024x over previous
import functools

import jax
import jax.numpy as jnp
from jax import lax
from jax.experimental import pallas as pl
from jax.experimental.pallas import tpu as pltpu


def _fused_kernel(x_ref, wqkv_ref, wo_ref, o_ref, wqkv_bf, wo_bf, *,
                  num_heads, dk, dv, L, eps, scale):
    nq = num_heads * dk

    @pl.when(pl.program_id(0) == 0)
    def _cast_weights():
        w = wqkv_ref[...]
        sc = jnp.where(
            lax.broadcasted_iota(jnp.int32, w.shape, 1) < nq, scale, 1.0)
        wqkv_bf[...] = (w * sc).astype(jnp.bfloat16)
        wo_bf[...] = wo_ref[...].astype(jnp.bfloat16)

    x = x_ref[0].astype(jnp.bfloat16)
    qkv = jnp.dot(x, wqkv_bf[...], preferred_element_type=jnp.float32)

    qkvb = qkv.astype(jnp.bfloat16)
    q = qkvb[:, :nq]
    k = qkvb[:, nq:2 * nq]
    v = qkvb[:, 2 * nq:]

    H2 = L // 2
    rowm = lax.broadcasted_iota(jnp.int32, (H2, H2), 0)
    colm = lax.broadcasted_iota(jnp.int32, (H2, H2), 1)
    tri = colm <= rowm
    causal_b = jnp.concatenate(
        [jnp.ones((H2, H2), jnp.bool_), tri], axis=-1)

    o_parts = []
    for h in range(num_heads):
        qh = q[:, h * dk:(h + 1) * dk]
        kh = k[:, h * dk:(h + 1) * dk]
        vh = v[:, h * dv:(h + 1) * dv]
        qa, qb = qh[:H2], qh[H2:]
        ka = kh[:H2]
        sa = lax.dot_general(qa, ka, (((1,), (1,)), ((), ())),
                             preferred_element_type=jnp.float32)
        attna = 1.0 + sa + 0.5 * (sa * sa)
        attna = jnp.where(tri, attna, 0.0)
        za = jnp.sum(attna, axis=-1, keepdims=True)
        oa = jnp.dot(attna.astype(jnp.bfloat16), vh[:H2],
                     preferred_element_type=jnp.float32)
        sb = lax.dot_general(qb, kh, (((1,), (1,)), ((), ())),
                             preferred_element_type=jnp.float32)
        attnb = 1.0 + sb + 0.5 * (sb * sb)
        attnb = jnp.where(causal_b, attnb, 0.0)
        zb = jnp.sum(attnb, axis=-1, keepdims=True)
        ob = jnp.dot(attnb.astype(jnp.bfloat16), vh,
                     preferred_element_type=jnp.float32)
        oh = jnp.concatenate(
            [oa * (1.0 / (za + eps)), ob * (1.0 / (zb + eps))], axis=0)
        o_parts.append(oh)
    o_norm = jnp.concatenate(o_parts, axis=-1).astype(jnp.bfloat16)

    o_ref[0] = jnp.dot(o_norm, wo_bf[...],
                       preferred_element_type=jnp.float32).astype(o_ref.dtype)


def kernel(Wqkv, Wo, x):
    B, L, D = x.shape
    num_heads = 8
    dk = 16
    nq = num_heads * dk
    nv = Wo.shape[0]
    dv = nv // num_heads
    eps = 1e-6
    scale = float(dk) ** -0.5

    body = functools.partial(_fused_kernel, num_heads=num_heads, dk=dk, dv=dv,
                             L=L, eps=eps, scale=scale)
    return pl.pallas_call(
        body,
        out_shape=jax.ShapeDtypeStruct((B, L, D), x.dtype),
        grid_spec=pltpu.PrefetchScalarGridSpec(
            num_scalar_prefetch=0,
            grid=(B,),
            in_specs=[
                pl.BlockSpec((1, L, D), lambda b: (b, 0, 0)),
                pl.BlockSpec((D, 2 * nq + nv), lambda b: (0, 0)),
                pl.BlockSpec((nv, D), lambda b: (0, 0)),
            ],
            out_specs=pl.BlockSpec((1, L, D), lambda b: (b, 0, 0)),
            scratch_shapes=[
                pltpu.VMEM((D, 2 * nq + nv), jnp.bfloat16),
                pltpu.VMEM((nv, D), jnp.bfloat16),
            ],
        ),
        compiler_params=pltpu.CompilerParams(
            dimension_semantics=("arbitrary",)),
    )(x, Wqkv, Wo)
